# baseline (speedup 1.0000x reference)
import functools

import jax
import jax.numpy as jnp
from jax import lax
from jax.experimental import pallas as pl
from jax.experimental.pallas import tpu as pltpu
from jax.experimental.pallas import tpu_sc as plsc

NC = 2
NS = 16
NW = NC * NS
CC = 64
NB = 8
LA = 4
NSEG = 2


def _deg_call(npad, rows_per_w):
    sl = npad // NS

    mesh = plsc.VectorSubcoreMesh(core_axis_name="c", subcore_axis_name="s")

    @functools.partial(
        pl.kernel,
        mesh=mesh,
        out_type=jax.ShapeDtypeStruct((NC * npad,), jnp.float32),
        scratch_types=[
            pltpu.VMEM((rows_per_w, CC), jnp.int32),
            pltpu.VMEM((CC,), jnp.float32),
            pltpu.VMEM((sl,), jnp.float32),
            pltpu.VMEM_SHARED((npad,), jnp.float32),
        ],
    )
    def deg_kernel(dst_hbm, deg_out, dstb, onesb, zb, deg_sh):
        cid = lax.axis_index("c")
        sid = lax.axis_index("s")
        z16 = jnp.zeros((16,), jnp.float32)
        o16 = jnp.ones((16,), jnp.float32)

        def zf(i, c):
            zb[pl.ds(i * 16, 16)] = z16
            return c
        lax.fori_loop(0, sl // 16, zf, 0)
        for k in range(CC // 16):
            onesb[pl.ds(k * 16, 16)] = o16

        base = sid * sl
        pltpu.sync_copy(zb, deg_sh.at[pl.ds(base, sl)])
        plsc.subcore_barrier()

        wid = cid * NS + sid
        pltpu.sync_copy(dst_hbm.at[pl.ds(wid * rows_per_w, rows_per_w)], dstb)

        def body(j, c):
            pltpu.sync_copy(onesb, deg_sh.at[dstb.at[j]], add=True)
            return c
        lax.fori_loop(0, rows_per_w, body, 0)
        plsc.subcore_barrier()
        pltpu.sync_copy(deg_sh.at[pl.ds(base, sl)],
                        deg_out.at[pl.ds(cid * npad + base, sl)])

    return deg_kernel


def _agg_call(npad, d, rows_per_w):
    sl = npad // NS

    mesh = plsc.VectorSubcoreMesh(core_axis_name="c", subcore_axis_name="s")

    @functools.partial(
        pl.kernel,
        mesh=mesh,
        out_type=jax.ShapeDtypeStruct((NC, npad, d), jnp.bfloat16),
        compiler_params=pltpu.CompilerParams(use_tc_tiling_on_sc=False),
        scratch_types=[
            pltpu.VMEM((rows_per_w // NSEG, CC), jnp.int32),
            pltpu.VMEM((rows_per_w // NSEG, CC), jnp.int32),
            pltpu.VMEM_SHARED((npad, d), jnp.bfloat16),
            pltpu.VMEM_SHARED((npad, d), jnp.bfloat16),
        ] + [pltpu.VMEM((CC, d), jnp.bfloat16) for _ in range(NB)]
          + [pltpu.SemaphoreType.DMA for _ in range(2 * NB)],
    )
    def agg_kernel(xs_hbm, src_hbm, dst_hbm, agg_out,
                   srcb, dstb, agg_sh, xs_sh, *ring):
        bufs, sg, ss = ring[0:NB], ring[NB:2 * NB], ring[2 * NB:3 * NB]
        cid = lax.axis_index("c")
        sid = lax.axis_index("s")
        z32 = jnp.zeros((32,), jnp.bfloat16)

        def zf(i, c):
            r = i // (d // 32)
            k = i % (d // 32)
            bufs[0][r, pl.ds(k * 32, 32)] = z32
            return c

        segrows = rows_per_w // NSEG
        base = sid * sl
        wid = cid * NS + sid
        wrow = wid * rows_per_w

        pltpu.sync_copy(xs_hbm.at[pl.ds(base, sl)], xs_sh.at[pl.ds(base, sl)])
        lax.fori_loop(0, CC * (d // 32), zf, 0)
        for k in range(sl // CC):
            pltpu.sync_copy(bufs[0], agg_sh.at[pl.ds(base + k * CC, CC), :])
        plsc.subcore_barrier()

        def g_start(j, b):
            pltpu.make_async_copy(xs_sh.at[srcb.at[j]], bufs[b], sg[b]).start()

        def g_wait(j, b):
            pltpu.make_async_copy(xs_sh.at[srcb.at[j]], bufs[b], sg[b]).wait()

        def s_start(j, b):
            pltpu.async_copy(bufs[b], agg_sh.at[dstb.at[j]], ss[b], add=True)

        def s_wait(j, b):
            pltpu.make_async_copy(bufs[b], agg_sh.at[dstb.at[j]], ss[b]).wait()

        for seg in range(NSEG):
            srow = wrow + seg * segrows
            pltpu.sync_copy(src_hbm.at[pl.ds(srow, segrows)], srcb)
            pltpu.sync_copy(dst_hbm.at[pl.ds(srow, segrows)], dstb)

            for b in range(LA):
                g_start(b, b)

            def body(g, c):
                for b in range(NB):
                    j = g * NB + b
                    bp = (b + LA) % NB
                    g_wait(j, b)
                    s_start(j, b)
                    nxt = j + LA

                    @pl.when(nxt < segrows)
                    def _():
                        @pl.when(j >= LA)
                        def _():
                            s_wait(j - LA, bp)
                        g_start(nxt, bp)
                return c
            lax.fori_loop(0, segrows // NB, body, 0)
            for b in range(NB):
                s_wait(segrows - NB + b, b)

        plsc.subcore_barrier()
        for k in range(sl // CC):
            pltpu.sync_copy(agg_sh.at[pl.ds(base + k * CC, CC), :],
                            agg_out.at[cid, pl.ds(base + k * CC, CC), :])

    return agg_kernel


def _prep_kernel(degp_ref, x_ref, dinv_ref, xs_ref, xsb_ref):
    deg = degp_ref[0] + degp_ref[1] + 1.0
    dinv = lax.rsqrt(deg)
    dinv_ref[...] = dinv
    xs = x_ref[...] * dinv
    xs_ref[...] = xs
    xsb_ref[...] = xs.astype(jnp.bfloat16)


def _out_kernel(agg_ref, xs_ref, dinv_ref, w_ref, b_ref, a_ref, o_ref):
    aggf = agg_ref[0].astype(jnp.float32) + agg_ref[1].astype(jnp.float32)
    pre = dinv_ref[...] * (aggf + xs_ref[...])
    h = jnp.dot(pre, w_ref[...], preferred_element_type=jnp.float32)
    h = h + b_ref[...]
    o_ref[...] = jnp.where(h >= 0.0, h, a_ref[...] * h)


def kernel(x, edge_index, W, b, alpha):
    n, din = x.shape
    e = edge_index.shape[1]
    dout = W.shape[1]
    assert din % 16 == 0
    npad = ((n + NS * 16 - 1) // (NS * 16)) * (NS * 16)
    assert npad > n
    epad = -(-e // (CC * NW * NB)) * (CC * NW * NB)
    rows = epad // CC
    rows_per_w = rows // NW

    pad_e = jnp.full((epad - e,), npad - 1, jnp.int32)
    src2 = jnp.concatenate([edge_index[0], pad_e]).reshape(rows, CC)
    dst2 = jnp.concatenate([edge_index[1], pad_e]).reshape(rows, CC)
    xpad = jnp.concatenate([x, jnp.zeros((npad - n, din), x.dtype)], axis=0)

    degp = _deg_call(npad, rows_per_w)(dst2)
    degp3 = degp.reshape(NC, npad, 1)

    dinv, xs, xsb = pl.pallas_call(
        _prep_kernel,
        out_shape=[
            jax.ShapeDtypeStruct((npad, 1), jnp.float32),
            jax.ShapeDtypeStruct((npad, din), jnp.float32),
            jax.ShapeDtypeStruct((npad, din), jnp.bfloat16),
        ],
    )(degp3, xpad)

    aggp = _agg_call(npad, din, rows_per_w)(xsb, src2, dst2)

    blk = 640
    grid = ((n + blk - 1) // blk,)
    out = pl.pallas_call(
        _out_kernel,
        grid=grid,
        in_specs=[
            pl.BlockSpec((NC, blk, din), lambda i: (0, i, 0)),
            pl.BlockSpec((blk, din), lambda i: (i, 0)),
            pl.BlockSpec((blk, 1), lambda i: (i, 0)),
            pl.BlockSpec((din, dout), lambda i: (0, 0)),
            pl.BlockSpec((1, dout), lambda i: (0, 0)),
            pl.BlockSpec((1, dout), lambda i: (0, 0)),
        ],
        out_specs=pl.BlockSpec((blk, dout), lambda i: (i, 0)),
        out_shape=jax.ShapeDtypeStruct((n, dout), jnp.float32),
    )(aggp, xs, dinv, W, b.reshape(1, dout), alpha.reshape(1, dout))
    return out


# speedup vs baseline: 1.0345x; 1.0345x over previous
import functools

import jax
import jax.numpy as jnp
from jax import lax
from jax.experimental import pallas as pl
from jax.experimental.pallas import tpu as pltpu
from jax.experimental.pallas import tpu_sc as plsc

NC = 2
NS = 16
NW = NC * NS
CC = 128
NB = 4
LA = 2
NSEG = 2


def _deg_call(npad, rows_per_w):
    sl = npad // NS

    mesh = plsc.VectorSubcoreMesh(core_axis_name="c", subcore_axis_name="s")

    @functools.partial(
        pl.kernel,
        mesh=mesh,
        out_type=jax.ShapeDtypeStruct((NC * npad,), jnp.float32),
        scratch_types=[
            pltpu.VMEM((rows_per_w, CC), jnp.int32),
            pltpu.VMEM((CC,), jnp.float32),
            pltpu.VMEM((sl,), jnp.float32),
            pltpu.VMEM_SHARED((npad,), jnp.float32),
        ],
    )
    def deg_kernel(dst_hbm, deg_out, dstb, onesb, zb, deg_sh):
        cid = lax.axis_index("c")
        sid = lax.axis_index("s")
        z16 = jnp.zeros((16,), jnp.float32)
        o16 = jnp.ones((16,), jnp.float32)

        def zf(i, c):
            zb[pl.ds(i * 16, 16)] = z16
            return c
        lax.fori_loop(0, sl // 16, zf, 0)
        for k in range(CC // 16):
            onesb[pl.ds(k * 16, 16)] = o16

        base = sid * sl
        pltpu.sync_copy(zb, deg_sh.at[pl.ds(base, sl)])
        plsc.subcore_barrier()

        wid = cid * NS + sid
        pltpu.sync_copy(dst_hbm.at[pl.ds(wid * rows_per_w, rows_per_w)], dstb)

        def body(j, c):
            pltpu.sync_copy(onesb, deg_sh.at[dstb.at[j]], add=True)
            return c
        lax.fori_loop(0, rows_per_w, body, 0)
        plsc.subcore_barrier()
        pltpu.sync_copy(deg_sh.at[pl.ds(base, sl)],
                        deg_out.at[pl.ds(cid * npad + base, sl)])

    return deg_kernel


def _agg_call(npad, d, rows_per_w):
    sl = npad // NS

    mesh = plsc.VectorSubcoreMesh(core_axis_name="c", subcore_axis_name="s")

    @functools.partial(
        pl.kernel,
        mesh=mesh,
        out_type=jax.ShapeDtypeStruct((NC, npad, d), jnp.bfloat16),
        compiler_params=pltpu.CompilerParams(use_tc_tiling_on_sc=False),
        scratch_types=[
            pltpu.VMEM((rows_per_w // NSEG, CC), jnp.int32),
            pltpu.VMEM((rows_per_w // NSEG, CC), jnp.int32),
            pltpu.VMEM_SHARED((npad, d), jnp.bfloat16),
            pltpu.VMEM_SHARED((npad, d), jnp.bfloat16),
        ] + [pltpu.VMEM((CC, d), jnp.bfloat16) for _ in range(NB)]
          + [pltpu.SemaphoreType.DMA for _ in range(2 * NB)],
    )
    def agg_kernel(xs_hbm, src_hbm, dst_hbm, agg_out,
                   srcb, dstb, agg_sh, xs_sh, *ring):
        bufs, sg, ss = ring[0:NB], ring[NB:2 * NB], ring[2 * NB:3 * NB]
        cid = lax.axis_index("c")
        sid = lax.axis_index("s")
        z32 = jnp.zeros((32,), jnp.bfloat16)

        def zf(i, c):
            r = i // (d // 32)
            k = i % (d // 32)
            bufs[0][r, pl.ds(k * 32, 32)] = z32
            return c

        segrows = rows_per_w // NSEG
        base = sid * sl
        wid = cid * NS + sid
        wrow = wid * rows_per_w

        pltpu.sync_copy(xs_hbm.at[pl.ds(base, sl)], xs_sh.at[pl.ds(base, sl)])
        lax.fori_loop(0, CC * (d // 32), zf, 0)
        for k in range(sl // CC):
            pltpu.sync_copy(bufs[0], agg_sh.at[pl.ds(base + k * CC, CC), :])
        plsc.subcore_barrier()

        def g_start(j, b):
            pltpu.make_async_copy(xs_sh.at[srcb.at[j]], bufs[b], sg[b]).start()

        def g_wait(j, b):
            pltpu.make_async_copy(xs_sh.at[srcb.at[j]], bufs[b], sg[b]).wait()

        def s_start(j, b):
            pltpu.async_copy(bufs[b], agg_sh.at[dstb.at[j]], ss[b], add=True)

        def s_wait(j, b):
            pltpu.make_async_copy(bufs[b], agg_sh.at[dstb.at[j]], ss[b]).wait()

        for seg in range(NSEG):
            srow = wrow + seg * segrows
            pltpu.sync_copy(src_hbm.at[pl.ds(srow, segrows)], srcb)
            pltpu.sync_copy(dst_hbm.at[pl.ds(srow, segrows)], dstb)

            for b in range(LA):
                g_start(b, b)

            def body(g, c):
                for b in range(NB):
                    j = g * NB + b
                    bp = (b + LA) % NB
                    g_wait(j, b)
                    s_start(j, b)
                    nxt = j + LA

                    @pl.when(nxt < segrows)
                    def _():
                        @pl.when(j >= LA)
                        def _():
                            s_wait(j - LA, bp)
                        g_start(nxt, bp)
                return c
            lax.fori_loop(0, segrows // NB, body, 0)
            for b in range(NB):
                s_wait(segrows - NB + b, b)

        plsc.subcore_barrier()
        for k in range(sl // CC):
            pltpu.sync_copy(agg_sh.at[pl.ds(base + k * CC, CC), :],
                            agg_out.at[cid, pl.ds(base + k * CC, CC), :])

    return agg_kernel


def _prep_kernel(degp_ref, x_ref, xsb_ref):
    deg = degp_ref[0] + degp_ref[1] + 1.0
    dinv = lax.rsqrt(deg)
    xsb_ref[...] = (x_ref[...] * dinv).astype(jnp.bfloat16)


def _out_kernel(agg_ref, degp_ref, x_ref, w_ref, b_ref, a_ref, o_ref):
    deg = degp_ref[0] + degp_ref[1] + 1.0
    dinv = lax.rsqrt(deg)
    aggf = agg_ref[0].astype(jnp.float32) + agg_ref[1].astype(jnp.float32)
    pre = dinv * aggf + (dinv * dinv) * x_ref[...]
    h = jnp.dot(pre, w_ref[...], preferred_element_type=jnp.float32)
    h = h + b_ref[...]
    o_ref[...] = jnp.where(h >= 0.0, h, a_ref[...] * h)


def kernel(x, edge_index, W, b, alpha):
    n, din = x.shape
    e = edge_index.shape[1]
    dout = W.shape[1]
    assert din % 16 == 0
    npad = ((n + NS * 16 - 1) // (NS * 16)) * (NS * 16)
    assert npad > n
    epad = -(-e // (CC * NW * NB)) * (CC * NW * NB)
    rows = epad // CC
    rows_per_w = rows // NW

    pad_e = jnp.full((epad - e,), npad - 1, jnp.int32)
    src2 = jnp.concatenate([edge_index[0], pad_e]).reshape(rows, CC)
    dst2 = jnp.concatenate([edge_index[1], pad_e]).reshape(rows, CC)
    xpad = jnp.concatenate([x, jnp.zeros((npad - n, din), x.dtype)], axis=0)

    degp = _deg_call(npad, rows_per_w)(dst2)
    degp3 = degp.reshape(NC, npad, 1)

    xsb = pl.pallas_call(
        _prep_kernel,
        out_shape=jax.ShapeDtypeStruct((npad, din), jnp.bfloat16),
    )(degp3, xpad)

    aggp = _agg_call(npad, din, rows_per_w)(xsb, src2, dst2)

    blk = 640
    grid = ((n + blk - 1) // blk,)
    out = pl.pallas_call(
        _out_kernel,
        grid=grid,
        in_specs=[
            pl.BlockSpec((NC, blk, din), lambda i: (0, i, 0)),
            pl.BlockSpec((NC, blk, 1), lambda i: (0, i, 0)),
            pl.BlockSpec((blk, din), lambda i: (i, 0)),
            pl.BlockSpec((din, dout), lambda i: (0, 0)),
            pl.BlockSpec((1, dout), lambda i: (0, 0)),
            pl.BlockSpec((1, dout), lambda i: (0, 0)),
        ],
        out_specs=pl.BlockSpec((blk, dout), lambda i: (i, 0)),
        out_shape=jax.ShapeDtypeStruct((n, dout), jnp.float32),
    )(aggp, degp3, xpad, W, b.reshape(1, dout), alpha.reshape(1, dout))
    return out
